# baseline (speedup 1.0000x reference)
import jax
import jax.numpy as jnp
import numpy as _np
from jax.experimental import pallas as pl

N = 512
INPUT_DIM = 32
EMBED_DIM = 64
HIDDEN = 64
LATENT = 32
PERIODS = 3

_EPS = _np.asarray(
    jax.random.normal(jax.random.key(42), (N, LATENT), jnp.float32))

_R_ATT = 0
_R_BIAS = 1
_R_ENTW = 16
_R_TIMW = 80
_R_ECZ = 144
_R_ECH = 304
_R_ELZ = 464
_R_ELH = 528
_R_MUW = 592
_R_LVW = 656
_R_DECW = 720
_R_DCZ = 752
_R_DCH = 816
_R_DLZ = 880
_R_DLH = 912
_ROWS = 944

_BIAS_ORDER = ('ent_b', 'time_b', 'e_conv_z_b', 'e_lin_z_b', 'e_conv_h_b',
               'e_lin_h_b', 'mu_b', 'lv_b', 'dec_b', 'd_conv_z_b',
               'd_lin_z_b', 'd_conv_h_b', 'd_lin_h_b')


def _fwd_kernel(x_ref, ent_ref, tim_ref, eps_ref, ws_ref, as_ref, pk_ref,
                recon_ref, mu_ref, lv_ref, w_ref, a_ref):
    ri = jax.lax.broadcasted_iota(jnp.int32, (N, N), 0)
    ci = jax.lax.broadcasted_iota(jnp.int32, (N, N), 1)
    W = jnp.where(ri == ci, 0.0, jax.nn.sigmoid(ws_ref[...]))
    A = jax.nn.sigmoid(as_ref[...])
    w_ref[...] = W
    a_ref[...] = A
    S = W + A

    ones = jnp.ones((N, 1), jnp.float32)
    deg = jax.lax.dot_general(
        S, ones, (((0,), (0,)), ((), ())), preferred_element_type=jnp.float32
    ) + 1.0
    dinv = jax.lax.rsqrt(deg)
    dinv2 = dinv * dinv

    def conv(y, Wg, bg):
        xw = jnp.dot(y, Wg, preferred_element_type=jnp.float32)
        v = dinv * xw
        u = jax.lax.dot_general(
            S, v, (((0,), (0,)), ((), ())), preferred_element_type=jnp.float32
        )
        return dinv * u + dinv2 * xw + bg

    def bias(idx, width=HIDDEN):
        return pk_ref[_R_BIAS + idx:_R_BIAS + idx + 1, :width]

    att = pk_ref[_R_ATT:_R_ATT + 1, :PERIODS]
    probs = jax.nn.softmax(att, axis=-1)

    entW = pk_ref[_R_ENTW:_R_ENTW + EMBED_DIM]
    timW = pk_ref[_R_TIMW:_R_TIMW + EMBED_DIM]
    ezW = pk_ref[_R_ECZ:_R_ECZ + INPUT_DIM + 2 * HIDDEN]
    ehW = pk_ref[_R_ECH:_R_ECH + INPUT_DIM + 2 * HIDDEN]
    elzW = pk_ref[_R_ELZ:_R_ELZ + HIDDEN]
    elhW = pk_ref[_R_ELH:_R_ELH + HIDDEN]
    entb, timb = bias(0), bias(1)
    ezb, elzb, ehb, elhb = bias(2), bias(3), bias(4), bias(5)

    Hacc = jnp.zeros((N, HIDDEN), jnp.float32)
    for t in range(PERIODS):
        xt = x_ref[0, t]
        ent_h = jax.nn.relu(
            jnp.dot(ent_ref[0, t], entW, preferred_element_type=jnp.float32)
            + entb)
        tim_h = jax.nn.relu(
            jnp.dot(tim_ref[0, t], timW, preferred_element_type=jnp.float32)
            + timb)
        h = jnp.concatenate([xt, ent_h, tim_h], axis=1)
        cz = conv(h, ezW, ezb)
        Z = jax.nn.sigmoid(
            jnp.dot(cz, elzW, preferred_element_type=jnp.float32) + elzb)
        ch = conv(h, ehW, ehb)
        Ht = jnp.tanh(
            jnp.dot(ch, elhW, preferred_element_type=jnp.float32) + elhb)
        Hacc = Hacc + probs[0, t] * ((1.0 - Z) * Ht)

    enc = jax.nn.relu(Hacc)
    muW = pk_ref[_R_MUW:_R_MUW + HIDDEN, :LATENT]
    lvW = pk_ref[_R_LVW:_R_LVW + HIDDEN, :LATENT]
    mu = jnp.dot(enc, muW, preferred_element_type=jnp.float32) \
        + bias(6, LATENT)
    lv = jnp.dot(enc, lvW, preferred_element_type=jnp.float32) \
        + bias(7, LATENT)
    mu_ref[...] = mu
    lv_ref[...] = lv
    z = mu + eps_ref[...] * jnp.exp(0.5 * lv)
    decW = pk_ref[_R_DECW:_R_DECW + LATENT]
    dh = jnp.dot(z, decW, preferred_element_type=jnp.float32) + bias(8)

    dzW = pk_ref[_R_DCZ:_R_DCZ + HIDDEN, :INPUT_DIM]
    dhW = pk_ref[_R_DCH:_R_DCH + HIDDEN, :INPUT_DIM]
    dlzW = pk_ref[_R_DLZ:_R_DLZ + INPUT_DIM, :INPUT_DIM]
    dlhW = pk_ref[_R_DLH:_R_DLH + INPUT_DIM, :INPUT_DIM]
    cz = conv(dh, dzW, bias(9, INPUT_DIM))
    Zd = jax.nn.sigmoid(
        jnp.dot(cz, dlzW, preferred_element_type=jnp.float32)
        + bias(10, INPUT_DIM))
    ch = conv(dh, dhW, bias(11, INPUT_DIM))
    Htd = jnp.tanh(
        jnp.dot(ch, dlhW, preferred_element_type=jnp.float32)
        + bias(12, INPUT_DIM))
    recon_ref[...] = jax.nn.relu((1.0 - Zd) * Htd)


def kernel(x, entity_emb, time_emb, num_nodes, params):
    p = params
    f32 = jnp.float32

    def pad64(m):
        m = jnp.asarray(m, f32)
        if m.ndim == 1:
            m = m[None, :]
        return jnp.pad(m, ((0, 0), (0, 64 - m.shape[1])))

    sections = [
        pad64(jnp.pad(p['att'][None, :], ((0, 0), (0, 0)))),
        jnp.concatenate([pad64(p[k]) for k in _BIAS_ORDER], axis=0),
        jnp.zeros((2, 64), f32),
        pad64(p['ent_W']), pad64(p['time_W']),
        pad64(p['e_conv_z_W']), pad64(p['e_conv_h_W']),
        pad64(p['e_lin_z_W'][:HIDDEN]), pad64(p['e_lin_h_W'][:HIDDEN]),
        pad64(p['mu_W']), pad64(p['lv_W']),
        pad64(p['dec_W']),
        pad64(p['d_conv_z_W']), pad64(p['d_conv_h_W']),
        pad64(p['d_lin_z_W'][:INPUT_DIM]), pad64(p['d_lin_h_W'][:INPUT_DIM]),
    ]
    packed = jnp.concatenate(sections, axis=0)

    out_shape = (
        jax.ShapeDtypeStruct((N, INPUT_DIM), f32),
        jax.ShapeDtypeStruct((N, LATENT), f32),
        jax.ShapeDtypeStruct((N, LATENT), f32),
        jax.ShapeDtypeStruct((N, N), f32),
        jax.ShapeDtypeStruct((N, N), f32),
    )
    return pl.pallas_call(_fwd_kernel, out_shape=out_shape)(
        x, entity_emb, time_emb, jnp.asarray(_EPS),
        p['W_score'], p['A_score'], packed)


# speedup vs baseline: 1.4642x; 1.4642x over previous
import jax
import jax.numpy as jnp
import numpy as _np
from jax.experimental import pallas as pl
from jax.experimental.pallas import tpu as pltpu

N = 512
INPUT_DIM = 32
EMBED_DIM = 64
HIDDEN = 64
LATENT = 32
PERIODS = 3

_EPS = _np.asarray(
    jax.random.normal(jax.random.key(42), (N, LATENT), jnp.float32))


def _fwd_kernel(
    x_hbm, ent_hbm, tim_hbm, eps_hbm,
    ws_ref, as_ref,
    entW_ref, entb_ref, timW_ref, timb_ref, att_ref,
    ezW_ref, ezb_ref, elzW_ref, elzb_ref,
    ehW_ref, ehb_ref, elhW_ref, elhb_ref,
    muW_ref, mub_ref, lvW_ref, lvb_ref,
    decW_ref, decb_ref,
    dzW_ref, dzb_ref, dlzW_ref, dlzb_ref,
    dhW_ref, dhb_ref, dlhW_ref, dlhb_ref,
    recon_ref, mu_ref, lv_ref, w_hbm, a_hbm,
    x_scr, ent_scr, tim_scr, eps_scr, w_scr, a_scr,
    sem_in, sem_w, sem_a,
):
    cp_x = pltpu.make_async_copy(x_hbm, x_scr, sem_in.at[0])
    cp_e = pltpu.make_async_copy(ent_hbm, ent_scr, sem_in.at[1])
    cp_t = pltpu.make_async_copy(tim_hbm, tim_scr, sem_in.at[2])
    cp_p = pltpu.make_async_copy(eps_hbm, eps_scr, sem_in.at[3])
    cp_x.start()
    cp_e.start()
    cp_t.start()
    cp_p.start()

    def rowvec(ref):
        return jnp.reshape(ref[...], (1, -1))

    ri = jax.lax.broadcasted_iota(jnp.int32, (N, N), 0)
    ci = jax.lax.broadcasted_iota(jnp.int32, (N, N), 1)
    W = jnp.where(ri == ci, 0.0, jax.nn.sigmoid(ws_ref[...]))
    w_scr[...] = W
    cp_w = pltpu.make_async_copy(w_scr, w_hbm, sem_w)
    cp_w.start()
    A = jax.nn.sigmoid(as_ref[...])
    a_scr[...] = A
    cp_a = pltpu.make_async_copy(a_scr, a_hbm, sem_a)
    cp_a.start()
    S = W + A

    ones = jnp.ones((N, 1), jnp.float32)
    deg = jax.lax.dot_general(
        S, ones, (((0,), (0,)), ((), ())), preferred_element_type=jnp.float32
    ) + 1.0
    dinv = jax.lax.rsqrt(deg)
    dinv2 = dinv * dinv

    def conv(y, Wg, bg):
        xw = jnp.dot(y, Wg, preferred_element_type=jnp.float32)
        v = dinv * xw
        u = jax.lax.dot_general(
            S, v, (((0,), (0,)), ((), ())), preferred_element_type=jnp.float32
        )
        return dinv * u + dinv2 * xw + bg

    probs = jax.nn.softmax(rowvec(att_ref), axis=-1)

    entW = entW_ref[...]
    entb = rowvec(entb_ref)
    timW = timW_ref[...]
    timb = rowvec(timb_ref)
    ezW = ezW_ref[...]
    ehW = ehW_ref[...]
    elzW = elzW_ref[:HIDDEN]
    elhW = elhW_ref[:HIDDEN]

    cp_x.wait()
    cp_e.wait()
    cp_t.wait()
    cp_p.wait()

    Hacc = jnp.zeros((N, HIDDEN), jnp.float32)
    for t in range(PERIODS):
        xt = x_scr[0, t]
        ent_h = jax.nn.relu(
            jnp.dot(ent_scr[0, t], entW, preferred_element_type=jnp.float32)
            + entb)
        tim_h = jax.nn.relu(
            jnp.dot(tim_scr[0, t], timW, preferred_element_type=jnp.float32)
            + timb)
        h = jnp.concatenate([xt, ent_h, tim_h], axis=1)
        cz = conv(h, ezW, rowvec(ezb_ref))
        Z = jax.nn.sigmoid(
            jnp.dot(cz, elzW, preferred_element_type=jnp.float32)
            + rowvec(elzb_ref))
        ch = conv(h, ehW, rowvec(ehb_ref))
        Ht = jnp.tanh(
            jnp.dot(ch, elhW, preferred_element_type=jnp.float32)
            + rowvec(elhb_ref))
        Hacc = Hacc + probs[0, t] * ((1.0 - Z) * Ht)

    enc = jax.nn.relu(Hacc)
    mu = jnp.dot(enc, muW_ref[...], preferred_element_type=jnp.float32) \
        + rowvec(mub_ref)
    lv = jnp.dot(enc, lvW_ref[...], preferred_element_type=jnp.float32) \
        + rowvec(lvb_ref)
    mu_ref[...] = mu
    lv_ref[...] = lv
    z = mu + eps_scr[...] * jnp.exp(0.5 * lv)
    dh = jnp.dot(z, decW_ref[...], preferred_element_type=jnp.float32) \
        + rowvec(decb_ref)

    cz = conv(dh, dzW_ref[...], rowvec(dzb_ref))
    Zd = jax.nn.sigmoid(
        jnp.dot(cz, dlzW_ref[:INPUT_DIM], preferred_element_type=jnp.float32)
        + rowvec(dlzb_ref))
    ch = conv(dh, dhW_ref[...], rowvec(dhb_ref))
    Htd = jnp.tanh(
        jnp.dot(ch, dlhW_ref[:INPUT_DIM], preferred_element_type=jnp.float32)
        + rowvec(dlhb_ref))
    recon_ref[...] = jax.nn.relu((1.0 - Zd) * Htd)

    cp_w.wait()
    cp_a.wait()


def kernel(x, entity_emb, time_emb, num_nodes, params):
    p = params
    f32 = jnp.float32
    operands = [
        x, entity_emb, time_emb, jnp.asarray(_EPS),
        p['W_score'], p['A_score'],
        p['ent_W'], p['ent_b'], p['time_W'], p['time_b'], p['att'],
        p['e_conv_z_W'], p['e_conv_z_b'], p['e_lin_z_W'], p['e_lin_z_b'],
        p['e_conv_h_W'], p['e_conv_h_b'], p['e_lin_h_W'], p['e_lin_h_b'],
        p['mu_W'], p['mu_b'], p['lv_W'], p['lv_b'],
        p['dec_W'], p['dec_b'],
        p['d_conv_z_W'], p['d_conv_z_b'], p['d_lin_z_W'], p['d_lin_z_b'],
        p['d_conv_h_W'], p['d_conv_h_b'], p['d_lin_h_W'], p['d_lin_h_b'],
    ]
    any_spec = pl.BlockSpec(memory_space=pl.MemorySpace.ANY)
    in_specs = [any_spec] * 4 + [pl.BlockSpec()] * (len(operands) - 4)
    out_specs = (pl.BlockSpec(), pl.BlockSpec(), pl.BlockSpec(),
                 any_spec, any_spec)
    out_shape = (
        jax.ShapeDtypeStruct((N, INPUT_DIM), f32),
        jax.ShapeDtypeStruct((N, LATENT), f32),
        jax.ShapeDtypeStruct((N, LATENT), f32),
        jax.ShapeDtypeStruct((N, N), f32),
        jax.ShapeDtypeStruct((N, N), f32),
    )
    return pl.pallas_call(
        _fwd_kernel,
        in_specs=in_specs,
        out_specs=out_specs,
        out_shape=out_shape,
        scratch_shapes=[
            pltpu.VMEM((1, PERIODS, N, INPUT_DIM), f32),
            pltpu.VMEM((1, PERIODS, N, EMBED_DIM), f32),
            pltpu.VMEM((1, PERIODS, N, EMBED_DIM), f32),
            pltpu.VMEM((N, LATENT), f32),
            pltpu.VMEM((N, N), f32),
            pltpu.VMEM((N, N), f32),
            pltpu.SemaphoreType.DMA((4,)),
            pltpu.SemaphoreType.DMA,
            pltpu.SemaphoreType.DMA,
        ],
    )(*operands)


# speedup vs baseline: 1.7486x; 1.1943x over previous
import jax
import jax.numpy as jnp
import numpy as _np
from jax.experimental import pallas as pl

N = 512
INPUT_DIM = 32
EMBED_DIM = 64
HIDDEN = 64
LATENT = 32
PERIODS = 3

_EPS = _np.asarray(
    jax.random.normal(jax.random.key(42), (N, LATENT), jnp.float32))


def _colsum_contract(a, b):
    return jax.lax.dot_general(
        a, b, (((0,), (0,)), ((), ())), preferred_element_type=jnp.float32)


def _mm(a, b):
    return jnp.dot(a, b, preferred_element_type=jnp.float32)


def _fwd_kernel(
    x_ref, ent_ref, tim_ref, eps_ref,
    ws_ref, as_ref,
    entW_ref, entb_ref, timW_ref, timb_ref, att_ref,
    ezW_ref, ezb_ref, elzW_ref, elzb_ref,
    ehW_ref, ehb_ref, elhW_ref, elhb_ref,
    muW_ref, mub_ref, lvW_ref, lvb_ref,
    decW_ref, decb_ref,
    dzW_ref, dzb_ref, dlzW_ref, dlzb_ref,
    dhW_ref, dhb_ref, dlhW_ref, dlhb_ref,
    recon_ref, mu_ref, lv_ref, w_ref, a_ref,
):
    def rowvec(ref):
        return jnp.reshape(ref[...], (1, -1))

    ri = jax.lax.broadcasted_iota(jnp.int32, (N, N), 0)
    ci = jax.lax.broadcasted_iota(jnp.int32, (N, N), 1)
    W = jnp.where(ri == ci, 0.0, jax.nn.sigmoid(ws_ref[...]))
    A = jax.nn.sigmoid(as_ref[...])
    w_ref[...] = W
    a_ref[...] = A
    S = W + A

    ones = jnp.ones((N, 1), jnp.float32)
    deg = _colsum_contract(S, ones) + 1.0
    dinv = jax.lax.rsqrt(deg)
    dinv2 = dinv * dinv

    probs = jax.nn.softmax(rowvec(att_ref), axis=-1)

    ent_all = jax.nn.relu(
        _mm(jnp.reshape(ent_ref[...], (PERIODS * N, EMBED_DIM)),
            entW_ref[...]) + rowvec(entb_ref))
    tim_all = jax.nn.relu(
        _mm(jnp.reshape(tim_ref[...], (PERIODS * N, EMBED_DIM)),
            timW_ref[...]) + rowvec(timb_ref))
    h_all = jnp.concatenate(
        [jnp.reshape(x_ref[...], (PERIODS * N, INPUT_DIM)), ent_all, tim_all],
        axis=1)

    WZH = jnp.concatenate([ezW_ref[...], ehW_ref[...]], axis=1)
    XW_all = _mm(h_all, WZH)
    V_all = jnp.concatenate([dinv, dinv, dinv], axis=0) * XW_all

    V = jnp.concatenate(
        [V_all[t * N:(t + 1) * N] for t in range(PERIODS)], axis=1)
    U = _colsum_contract(S, V)

    bzh = jnp.concatenate([rowvec(ezb_ref), rowvec(ehb_ref)], axis=1)
    zeros_hh = jnp.zeros((HIDDEN, HIDDEN), jnp.float32)
    BD = jnp.concatenate([
        jnp.concatenate([elzW_ref[:HIDDEN], zeros_hh], axis=1),
        jnp.concatenate([zeros_hh, elhW_ref[:HIDDEN]], axis=1)], axis=0)
    blz = rowvec(elzb_ref)
    blh = rowvec(elhb_ref)

    Hacc = jnp.zeros((N, HIDDEN), jnp.float32)
    for t in range(PERIODS):
        xw_t = XW_all[t * N:(t + 1) * N]
        c_t = dinv * U[:, t * 128:(t + 1) * 128] + dinv2 * xw_t + bzh
        G = _mm(c_t, BD)
        Z = jax.nn.sigmoid(G[:, :HIDDEN] + blz)
        Ht = jnp.tanh(G[:, HIDDEN:] + blh)
        Hacc = Hacc + probs[0, t] * ((1.0 - Z) * Ht)

    enc = jax.nn.relu(Hacc)
    mulvW = jnp.concatenate([muW_ref[...], lvW_ref[...]], axis=1)
    mulvb = jnp.concatenate([rowvec(mub_ref), rowvec(lvb_ref)], axis=1)
    mulv = _mm(enc, mulvW) + mulvb
    mu = mulv[:, :LATENT]
    lv = mulv[:, LATENT:]
    mu_ref[...] = mu
    lv_ref[...] = lv
    z = mu + eps_ref[...] * jnp.exp(0.5 * lv)
    dh = _mm(z, decW_ref[...]) + rowvec(decb_ref)

    WZH_d = jnp.concatenate([dzW_ref[...], dhW_ref[...]], axis=1)
    xw_d = _mm(dh, WZH_d)
    u_d = _colsum_contract(S, dinv * xw_d)
    bzh_d = jnp.concatenate([rowvec(dzb_ref), rowvec(dhb_ref)], axis=1)
    c_d = dinv * u_d + dinv2 * xw_d + bzh_d
    zeros_ii = jnp.zeros((INPUT_DIM, INPUT_DIM), jnp.float32)
    BD_d = jnp.concatenate([
        jnp.concatenate([dlzW_ref[:INPUT_DIM], zeros_ii], axis=1),
        jnp.concatenate([zeros_ii, dlhW_ref[:INPUT_DIM]], axis=1)], axis=0)
    G_d = _mm(c_d, BD_d)
    Zd = jax.nn.sigmoid(G_d[:, :INPUT_DIM] + rowvec(dlzb_ref))
    Htd = jnp.tanh(G_d[:, INPUT_DIM:] + rowvec(dlhb_ref))
    recon_ref[...] = jax.nn.relu((1.0 - Zd) * Htd)


def kernel(x, entity_emb, time_emb, num_nodes, params):
    p = params
    f32 = jnp.float32
    operands = [
        x, entity_emb, time_emb, jnp.asarray(_EPS),
        p['W_score'], p['A_score'],
        p['ent_W'], p['ent_b'], p['time_W'], p['time_b'], p['att'],
        p['e_conv_z_W'], p['e_conv_z_b'], p['e_lin_z_W'], p['e_lin_z_b'],
        p['e_conv_h_W'], p['e_conv_h_b'], p['e_lin_h_W'], p['e_lin_h_b'],
        p['mu_W'], p['mu_b'], p['lv_W'], p['lv_b'],
        p['dec_W'], p['dec_b'],
        p['d_conv_z_W'], p['d_conv_z_b'], p['d_lin_z_W'], p['d_lin_z_b'],
        p['d_conv_h_W'], p['d_conv_h_b'], p['d_lin_h_W'], p['d_lin_h_b'],
    ]
    out_shape = (
        jax.ShapeDtypeStruct((N, INPUT_DIM), f32),
        jax.ShapeDtypeStruct((N, LATENT), f32),
        jax.ShapeDtypeStruct((N, LATENT), f32),
        jax.ShapeDtypeStruct((N, N), f32),
        jax.ShapeDtypeStruct((N, N), f32),
    )
    return pl.pallas_call(_fwd_kernel, out_shape=out_shape)(*operands)


# speedup vs baseline: 1.8049x; 1.0322x over previous
import jax
import jax.numpy as jnp
import numpy as _np
from jax.experimental import pallas as pl

N = 512
INPUT_DIM = 32
EMBED_DIM = 64
HIDDEN = 64
LATENT = 32
PERIODS = 3

_EPS = _np.asarray(
    jax.random.normal(jax.random.key(42), (N, LATENT), jnp.float32))

_M64_ENTW = 0
_M64_TIMW = 64
_M64_EZW = 128
_M64_EHW = 288
_M64_ELZ = 448
_M64_ELH = 576
_M64_DECW = 704
_M64_ROWS = 736

_M32_MUW = 0
_M32_LVW = 64
_M32_DZW = 128
_M32_DHW = 192
_M32_DLZ = 256
_M32_DLH = 320
_M32_ROWS = 384

_B_ENT, _B_TIM, _B_ECZ, _B_ELZ, _B_ECH, _B_ELH = 0, 64, 128, 192, 256, 320
_B_MU, _B_LV, _B_DEC = 384, 416, 448
_B_DCZ, _B_DLZ, _B_DCH, _B_DLH, _B_ATT = 512, 544, 576, 608, 640


def _colsum_contract(a, b):
    return jax.lax.dot_general(
        a, b, (((0,), (0,)), ((), ())), preferred_element_type=jnp.float32)


def _mm(a, b):
    return jnp.dot(a, b, preferred_element_type=jnp.float32)


def _fwd_kernel(
    x_ref, ent_ref, tim_ref, eps_ref,
    ws_ref, as_ref, m64_ref, m32_ref, bias_ref,
    recon_ref, mu_ref, lv_ref, w_ref, a_ref,
):
    def bias(off, width=HIDDEN):
        return bias_ref[0:1, off:off + width]

    ri = jax.lax.broadcasted_iota(jnp.int32, (N, N), 0)
    ci = jax.lax.broadcasted_iota(jnp.int32, (N, N), 1)
    W = jnp.where(ri == ci, 0.0, jax.nn.sigmoid(ws_ref[...]))
    A = jax.nn.sigmoid(as_ref[...])
    w_ref[...] = W
    a_ref[...] = A
    S = W + A

    ones = jnp.ones((N, 1), jnp.float32)
    deg = _colsum_contract(S, ones) + 1.0
    dinv = jax.lax.rsqrt(deg)
    dinv2 = dinv * dinv

    probs = jax.nn.softmax(bias(_B_ATT, PERIODS), axis=-1)

    ent_all = jax.nn.relu(
        _mm(jnp.reshape(ent_ref[...], (PERIODS * N, EMBED_DIM)),
            m64_ref[_M64_ENTW:_M64_ENTW + EMBED_DIM]) + bias(_B_ENT))
    tim_all = jax.nn.relu(
        _mm(jnp.reshape(tim_ref[...], (PERIODS * N, EMBED_DIM)),
            m64_ref[_M64_TIMW:_M64_TIMW + EMBED_DIM]) + bias(_B_TIM))
    h_all = jnp.concatenate(
        [jnp.reshape(x_ref[...], (PERIODS * N, INPUT_DIM)), ent_all, tim_all],
        axis=1)

    cin = INPUT_DIM + 2 * HIDDEN
    WZH = jnp.concatenate([
        m64_ref[_M64_EZW:_M64_EZW + cin],
        m64_ref[_M64_EHW:_M64_EHW + cin]], axis=1)
    XW_all = _mm(h_all, WZH)
    V_all = jnp.concatenate([dinv, dinv, dinv], axis=0) * XW_all

    V = jnp.concatenate(
        [V_all[t * N:(t + 1) * N] for t in range(PERIODS)], axis=1)
    U = _colsum_contract(S, V)

    bzh = jnp.concatenate([bias(_B_ECZ), bias(_B_ECH)], axis=1)
    zeros_hh = jnp.zeros((HIDDEN, HIDDEN), jnp.float32)
    BD = jnp.concatenate([
        jnp.concatenate([m64_ref[_M64_ELZ:_M64_ELZ + HIDDEN], zeros_hh],
                        axis=1),
        jnp.concatenate([zeros_hh, m64_ref[_M64_ELH:_M64_ELH + HIDDEN]],
                        axis=1)], axis=0)
    blz = bias(_B_ELZ)
    blh = bias(_B_ELH)

    Hacc = jnp.zeros((N, HIDDEN), jnp.float32)
    for t in range(PERIODS):
        xw_t = XW_all[t * N:(t + 1) * N]
        c_t = dinv * U[:, t * 128:(t + 1) * 128] + dinv2 * xw_t + bzh
        G = _mm(c_t, BD)
        Z = jax.nn.sigmoid(G[:, :HIDDEN] + blz)
        Ht = jnp.tanh(G[:, HIDDEN:] + blh)
        Hacc = Hacc + probs[0, t] * ((1.0 - Z) * Ht)

    enc = jax.nn.relu(Hacc)
    mulvW = jnp.concatenate([
        m32_ref[_M32_MUW:_M32_MUW + HIDDEN],
        m32_ref[_M32_LVW:_M32_LVW + HIDDEN]], axis=1)
    mulvb = jnp.concatenate(
        [bias(_B_MU, LATENT), bias(_B_LV, LATENT)], axis=1)
    mulv = _mm(enc, mulvW) + mulvb
    mu = mulv[:, :LATENT]
    lv = mulv[:, LATENT:]
    mu_ref[...] = mu
    lv_ref[...] = lv
    z = mu + eps_ref[...] * jnp.exp(0.5 * lv)
    dh = _mm(z, m64_ref[_M64_DECW:_M64_DECW + LATENT]) + bias(_B_DEC)

    WZH_d = jnp.concatenate([
        m32_ref[_M32_DZW:_M32_DZW + HIDDEN],
        m32_ref[_M32_DHW:_M32_DHW + HIDDEN]], axis=1)
    xw_d = _mm(dh, WZH_d)
    u_d = _colsum_contract(S, dinv * xw_d)
    bzh_d = jnp.concatenate(
        [bias(_B_DCZ, INPUT_DIM), bias(_B_DCH, INPUT_DIM)], axis=1)
    c_d = dinv * u_d + dinv2 * xw_d + bzh_d
    zeros_ii = jnp.zeros((INPUT_DIM, INPUT_DIM), jnp.float32)
    BD_d = jnp.concatenate([
        jnp.concatenate([m32_ref[_M32_DLZ:_M32_DLZ + INPUT_DIM], zeros_ii],
                        axis=1),
        jnp.concatenate([zeros_ii, m32_ref[_M32_DLH:_M32_DLH + INPUT_DIM]],
                        axis=1)], axis=0)
    G_d = _mm(c_d, BD_d)
    Zd = jax.nn.sigmoid(G_d[:, :INPUT_DIM] + bias(_B_DLZ, INPUT_DIM))
    Htd = jnp.tanh(G_d[:, INPUT_DIM:] + bias(_B_DLH, INPUT_DIM))
    recon_ref[...] = jax.nn.relu((1.0 - Zd) * Htd)


def kernel(x, entity_emb, time_emb, num_nodes, params):
    p = params
    f32 = jnp.float32
    m64 = jnp.concatenate([
        p['ent_W'], p['time_W'], p['e_conv_z_W'], p['e_conv_h_W'],
        p['e_lin_z_W'], p['e_lin_h_W'], p['dec_W']], axis=0)
    m32 = jnp.concatenate([
        p['mu_W'], p['lv_W'], p['d_conv_z_W'], p['d_conv_h_W'],
        p['d_lin_z_W'], p['d_lin_h_W']], axis=0)
    biases = jnp.concatenate([
        p['ent_b'], p['time_b'], p['e_conv_z_b'], p['e_lin_z_b'],
        p['e_conv_h_b'], p['e_lin_h_b'], p['mu_b'], p['lv_b'], p['dec_b'],
        p['d_conv_z_b'], p['d_lin_z_b'], p['d_conv_h_b'], p['d_lin_h_b'],
        p['att']])[None, :]
    operands = [
        x, entity_emb, time_emb, jnp.asarray(_EPS),
        p['W_score'], p['A_score'], m64, m32, biases,
    ]
    out_shape = (
        jax.ShapeDtypeStruct((N, INPUT_DIM), f32),
        jax.ShapeDtypeStruct((N, LATENT), f32),
        jax.ShapeDtypeStruct((N, LATENT), f32),
        jax.ShapeDtypeStruct((N, N), f32),
        jax.ShapeDtypeStruct((N, N), f32),
    )
    return pl.pallas_call(_fwd_kernel, out_shape=out_shape)(*operands)
